# Initial kernel scaffold; baseline (speedup 1.0000x reference)
import jax
import jax.numpy as jnp
from jax.experimental import pallas as pl


def kernel(pc, params, consts):
    raise NotImplementedError("write your pallas kernel here")


# speedup vs baseline: 3.8710x; 3.8710x over previous
import functools

import numpy as np
import jax
import jax.numpy as jnp
from jax import lax
from jax.experimental import pallas as pl
from jax.experimental.pallas import tpu as pltpu
from jax.experimental.pallas import tpu_sc as plsc

_N = [100000, 50000, 25000, 12500, 6250]
_LL = [50000, 25000, 12500, 6250, 3125]
_CIN = [36, 36, 68, 132, 260]
_COUT = [32, 64, 128, 256, 256]
_PAD = [100352, 50176, 25088, 12800, 6656, 3584]
_NBR = 9
_BL = 512
_CHUNK = 128
_NW = 32

_COMBOS = np.array([[sx, sy, sz]
                    for sx in (1.0, -1.0)
                    for sy in (1.0, -1.0)
                    for sz in (1.0, -1.0)], dtype=np.float32)


def _leaky(x):
    return jnp.where(x > 0, x, 0.1 * x)


def _mm(a, b):
    return lax.dot_general(a, b, (((1,), (0,)), ((), ())),
                           precision=lax.Precision.DEFAULT,
                           preferred_element_type=jnp.float32)


def _full(shape):
    return pl.BlockSpec(shape, lambda i: tuple(0 for _ in shape))


def _pre_call(pT, elT, baryT, w1, b1, w2, b2, w3, b3):
    Np = pT.shape[0]
    C = 36

    def body(p_ref, w1_ref, b1_ref, w2_ref, b2_ref, w3_ref, b3_ref,
             el_ref, ba_ref, out_ref):
        h = _leaky(_mm(p_ref[...], w1_ref[...]) + b1_ref[...])
        h = _leaky(_mm(h, w2_ref[...]) + b2_ref[...])
        h = _leaky(_mm(h, w3_ref[...]) + b3_ref[...])
        xc = jnp.concatenate([el_ref[...], h], axis=1)
        ba = ba_ref[...]
        for k in range(5):
            out_ref[k, :, :] = ba[:, k:k + 1] * xc

    return pl.pallas_call(
        body,
        grid=(Np // _BL,),
        in_specs=[
            pl.BlockSpec((_BL, 3), lambda i: (i, 0)),
            _full((3, 32)), _full((1, 32)),
            _full((32, 32)), _full((1, 32)),
            _full((32, 32)), _full((1, 32)),
            pl.BlockSpec((_BL, 4), lambda i: (i, 0)),
            pl.BlockSpec((_BL, 5), lambda i: (i, 0)),
        ],
        out_specs=pl.BlockSpec((5, _BL, C), lambda i: (0, i, 0)),
        out_shape=jax.ShapeDtypeStruct((5, Np, C), jnp.float32),
    )(pT, w1, b1, w2, b2, w3, b3, elT, baryT)


def _stage_call(G, w1t, b1, w2t, b2, elT, baryT):
    nine, Lp, C = G.shape
    cout = w1t.shape[2]

    def body(g_ref, w1_ref, b1_ref, w2_ref, b2_ref, el_ref, ba_ref, out_ref):
        g = g_ref[...]
        w1 = w1_ref[...]
        acc = _mm(g[0], w1[0])
        for j in range(1, _NBR):
            acc = acc + _mm(g[j], w1[j])
        h = _leaky(acc + b1_ref[...])
        h = _leaky(_mm(h, w2_ref[...]) + b2_ref[...])
        xc = jnp.concatenate([el_ref[...], h], axis=1)
        ba = ba_ref[...]
        for k in range(5):
            out_ref[k, :, :] = ba[:, k:k + 1] * xc

    return pl.pallas_call(
        body,
        grid=(Lp // _BL,),
        in_specs=[
            pl.BlockSpec((_NBR, _BL, C), lambda i: (0, i, 0)),
            _full((_NBR, C, cout)), _full((1, cout)),
            _full((cout, cout)), _full((1, cout)),
            pl.BlockSpec((_BL, 4), lambda i: (i, 0)),
            pl.BlockSpec((_BL, 5), lambda i: (i, 0)),
        ],
        out_specs=pl.BlockSpec((5, _BL, cout + 4), lambda i: (0, i, 0)),
        out_shape=jax.ShapeDtypeStruct((5, Lp, cout + 4), jnp.float32),
    )(G, w1t, b1, w2t, b2, elT, baryT)


def _stage4_call(G, w1t, b1, w2t, b2):
    nine, Lp, C = G.shape
    cout = w1t.shape[2]

    def body(g_ref, w1_ref, b1_ref, w2_ref, b2_ref, out_ref):
        g = g_ref[...]
        w1 = w1_ref[...]
        acc = _mm(g[0], w1[0])
        for j in range(1, _NBR):
            acc = acc + _mm(g[j], w1[j])
        h = _leaky(acc + b1_ref[...])
        out_ref[...] = _leaky(_mm(h, w2_ref[...]) + b2_ref[...])

    return pl.pallas_call(
        body,
        grid=(Lp // _BL,),
        in_specs=[
            pl.BlockSpec((_NBR, _BL, C), lambda i: (0, i, 0)),
            _full((_NBR, C, cout)), _full((1, cout)),
            _full((cout, cout)), _full((1, cout)),
        ],
        out_specs=pl.BlockSpec((_BL, cout), lambda i: (i, 0)),
        out_shape=jax.ShapeDtypeStruct((Lp, cout), jnp.float32),
    )(G, w1t, b1, w2t, b2)


def _sum_call(P):
    _, Lp, C = P.shape

    def body(p_ref, o_ref):
        o_ref[...] = p_ref[0] + p_ref[1]

    return pl.pallas_call(
        body,
        grid=(Lp // _BL,),
        in_specs=[pl.BlockSpec((2, _BL, C), lambda i: (0, i, 0))],
        out_specs=pl.BlockSpec((_BL, C), lambda i: (i, 0)),
        out_shape=jax.ShapeDtypeStruct((Lp, C), jnp.float32),
    )(P)


def _head_call(h4, nval, wg1, bg1, wg2, bg2, wg3, bg3,
               wl1, bl1, wl2, bl2, wl3, bl3,
               wabs, babs, wsgn, bsgn, combos):
    Lp = h4.shape[0]

    def body(h_ref, wg1_ref, bg1_ref, wg2_ref, bg2_ref, wg3_ref, bg3_ref,
             wl1_ref, bl1_ref, wl2_ref, bl2_ref, wl3_ref, bl3_ref,
             wabs_ref, babs_ref, wsgn_ref, bsgn_ref, cmb_ref, out_ref):
        g = jax.nn.relu(_mm(h_ref[...], wg1_ref[...]) + bg1_ref[...])
        g = jax.nn.relu(_mm(g, wg2_ref[...]) + bg2_ref[...])
        g = jax.nn.relu(_mm(g, wg3_ref[...]) + bg3_ref[...])
        rid = lax.broadcasted_iota(jnp.int32, (Lp, 128), 0)
        g = jnp.where(rid < nval, g, -1e30)
        fc0 = jnp.max(g, axis=0, keepdims=True)
        fc1 = jax.nn.relu(_mm(fc0, wl1_ref[...]) + bl1_ref[...])
        fc2 = jax.nn.relu(_mm(fc1, wl2_ref[...]) + bl2_ref[...])
        fc3 = jax.nn.relu(_mm(fc2, wl3_ref[...]) + bl3_ref[...])
        sgn = _mm(fc3, wsgn_ref[...]) + bsgn_ref[...]
        ab = _mm(fc3, wabs_ref[...]) + babs_ref[...]
        ab = ab - jnp.max(ab, axis=1, keepdims=True)
        eab = jnp.exp(ab)
        abs0 = eab / jnp.sum(eab, axis=1, keepdims=True)
        denom = jnp.sqrt(jnp.sum(abs0 * abs0, axis=1, keepdims=True))
        gn_abs3 = abs0 / denom
        sm = sgn - jnp.max(sgn, axis=1, keepdims=True)
        esm = jnp.exp(sm)
        sp = esm / jnp.sum(esm, axis=1, keepdims=True)
        signs = _mm(sp, cmb_ref[...])
        e = gn_abs3 * signs
        e_gn = e / jnp.sqrt(jnp.sum(e * e, axis=1, keepdims=True))
        out_ref[...] = jnp.concatenate([gn_abs3, sgn, e_gn], axis=1)

    return pl.pallas_call(
        body,
        grid=(1,),
        in_specs=[
            _full((Lp, 256)),
            _full((256, 128)), _full((1, 128)),
            _full((128, 128)), _full((1, 128)),
            _full((128, 128)), _full((1, 128)),
            _full((128, 128)), _full((1, 128)),
            _full((128, 128)), _full((1, 128)),
            _full((128, 32)), _full((1, 32)),
            _full((32, 3)), _full((1, 3)),
            _full((32, 8)), _full((1, 8)),
            _full((8, 3)),
        ],
        out_specs=_full((1, 14)),
        out_shape=jax.ShapeDtypeStruct((1, 14), jnp.float32),
    )(h4, wg1, bg1, wg2, bg2, wg3, bg3, wl1, bl1, wl2, bl2, wl3, bl3,
      wabs, babs, wsgn, bsgn, combos)


def _splat_call(vals, idx, Lp, C):
    R = vals.shape[0]
    tot = R // _CHUNK
    per = -(-tot // _NW)
    zr = Lp // 16
    zeros = jnp.zeros((zr, C), jnp.float32)
    mesh = plsc.VectorSubcoreMesh(core_axis_name="c", subcore_axis_name="s")

    @functools.partial(
        pl.kernel, mesh=mesh,
        compiler_params=pltpu.CompilerParams(use_tc_tiling_on_sc=False),
        out_type=jax.ShapeDtypeStruct((2, Lp, C), jnp.float32),
        scratch_types=[
            pltpu.VMEM((1, _CHUNK), jnp.int32),
            pltpu.VMEM((_CHUNK, C), jnp.float32),
            pltpu.VMEM_SHARED((Lp, C), jnp.float32),
            pltpu.SemaphoreType.DMA,
        ])
    def k(vals_hbm, idx_hbm, z_hbm, out_hbm, idx_v, vals_v, tab_sh, sem):
        cid = lax.axis_index("c")
        sid = lax.axis_index("s")
        wid = sid * 2 + cid
        pltpu.sync_copy(z_hbm, tab_sh.at[pl.ds(sid * zr, zr)])
        plsc.subcore_barrier()

        @pl.loop(0, per)
        def _(i):
            ch = wid * per + i

            @pl.when(ch < tot)
            def _():
                base = ch * _CHUNK
                pltpu.sync_copy(idx_hbm.at[pl.ds(base, _CHUNK)], idx_v.at[0])
                pltpu.sync_copy(vals_hbm.at[pl.ds(base, _CHUNK)], vals_v)
                pltpu.sync_copy(vals_v, tab_sh.at[idx_v.at[0]], add=True)

        plsc.subcore_barrier()
        pltpu.sync_copy(tab_sh.at[pl.ds(sid * zr, zr)],
                        out_hbm.at[cid, pl.ds(sid * zr, zr)])

    return k(vals, idx, zeros)


def _gather_call(S, nbr_flat, Lp, C):
    tot = (_NBR * Lp) // _CHUNK
    per = -(-tot // _NW)
    cb = Lp // _CHUNK
    mesh = plsc.VectorSubcoreMesh(core_axis_name="c", subcore_axis_name="s")

    @functools.partial(
        pl.kernel, mesh=mesh,
        compiler_params=pltpu.CompilerParams(use_tc_tiling_on_sc=False),
        out_type=jax.ShapeDtypeStruct((_NBR, Lp, C), jnp.float32),
        scratch_types=[
            pltpu.VMEM((_CHUNK,), jnp.int32),
            pltpu.VMEM((_CHUNK, C), jnp.float32),
            pltpu.SemaphoreType.DMA,
        ])
    def k(s_hbm, nbr_hbm, out_hbm, idx_v, rows_v, sem):
        wid = lax.axis_index("s") * 2 + lax.axis_index("c")

        @pl.loop(0, per)
        def _(i):
            ch = wid * per + i

            @pl.when(ch < tot)
            def _():
                j = ch // cb
                l0 = (ch % cb) * _CHUNK
                pltpu.sync_copy(nbr_hbm.at[pl.ds(ch * _CHUNK, _CHUNK)], idx_v)
                pltpu.async_copy(s_hbm.at[idx_v], rows_v, sem).wait()
                pltpu.sync_copy(rows_v, out_hbm.at[j, pl.ds(l0, _CHUNK)])

    return k(S, nbr_flat)


def kernel(pc, params, consts):
    f32 = jnp.float32
    p = pc[0].astype(f32)
    pT = jnp.pad(p.T, ((0, _PAD[0] - _N[0]), (0, 0)))

    elT, baryT, offf, nbrf = [], [], [], []
    for s in range(5):
        N, L = _N[s], _LL[s]
        Np, Lp = _PAD[s], _PAD[s + 1]
        elT.append(jnp.pad(consts['el_%d' % s].astype(f32).T,
                           ((0, Np - N), (0, 0))))
        baryT.append(jnp.pad(consts['bary_%d' % s].astype(f32).T,
                             ((0, Np - N), (0, 0))))
        off = jnp.pad(consts['off_%d' % s].astype(jnp.int32),
                      ((0, 0), (0, Np - N)), constant_values=L)
        offf.append(off.reshape(-1))
        nbr = jnp.pad(consts['nbr_%d' % s].astype(jnp.int32),
                      ((0, 0), (0, Lp - L)))
        nbrf.append(nbr.reshape(-1))

    w1t = [params['W1_%d' % s].astype(f32)
           .reshape(_COUT[s], _CIN[s], _NBR).transpose(2, 1, 0)
           for s in range(5)]
    w2t = [params['W2_%d' % s].astype(f32).T for s in range(5)]
    b1 = [params['b1_%d' % s].astype(f32).reshape(1, -1) for s in range(5)]
    b2 = [params['b2_%d' % s].astype(f32).reshape(1, -1) for s in range(5)]

    vals = _pre_call(pT, elT[0], baryT[0],
                     params['Wc1'].astype(f32).T,
                     params['bc1'].astype(f32).reshape(1, -1),
                     params['Wc2'].astype(f32).T,
                     params['bc2'].astype(f32).reshape(1, -1),
                     params['Wc3'].astype(f32).T,
                     params['bc3'].astype(f32).reshape(1, -1))

    h4 = None
    for s in range(5):
        C = _CIN[s]
        Lp = _PAD[s + 1]
        partials = _splat_call(vals.reshape(-1, C), offf[s], Lp, C)
        S = _sum_call(partials)
        G = _gather_call(S, nbrf[s], Lp, C)
        if s < 4:
            vals = _stage_call(G, w1t[s], b1[s], w2t[s], b2[s],
                               elT[s + 1], baryT[s + 1])
        else:
            h4 = _stage4_call(G, w1t[s], b1[s], w2t[s], b2[s])

    out = _head_call(
        h4, _LL[4],
        params['Wg1'].astype(f32).T, params['bg1'].astype(f32).reshape(1, -1),
        params['Wg2'].astype(f32).T, params['bg2'].astype(f32).reshape(1, -1),
        params['Wg3'].astype(f32).T, params['bg3'].astype(f32).reshape(1, -1),
        params['Wl1'].astype(f32).T, params['bl1'].astype(f32).reshape(1, -1),
        params['Wl2'].astype(f32).T, params['bl2'].astype(f32).reshape(1, -1),
        params['Wl3'].astype(f32).T, params['bl3'].astype(f32).reshape(1, -1),
        params['Wabs'].astype(f32).T, params['babs'].astype(f32).reshape(1, -1),
        params['Wsgn'].astype(f32).T, params['bsgn'].astype(f32).reshape(1, -1),
        jnp.asarray(_COMBOS))

    gn_abs3 = out[0, 0:3]
    gn_sgn = out[0, 3:11].reshape(1, 8)
    e_gn = out[0, 11:14]

    gn_abs = gn_abs3.reshape(1, 3, 1)
    a = e_gn
    bvec = jnp.array([0.0, 0.0, 1.0], f32)
    v = jnp.cross(a, bvec)
    c = jnp.dot(a, bvec)
    zero = jnp.float32(0.0)
    K = jnp.stack([
        jnp.stack([zero, -v[2], v[1]]),
        jnp.stack([v[2], zero, -v[0]]),
        jnp.stack([-v[1], v[0], zero]),
    ])
    R = jnp.eye(3, dtype=f32) + K + (K @ K) * (1.0 / (1.0 + c + 1e-8))
    return (gn_abs, gn_sgn, e_gn.reshape(1, 3), R[None, :, :])
